# baseline (speedup 1.0000x reference)
import functools

import jax
import jax.numpy as jnp
from jax import lax
from jax.experimental import pallas as pl
from jax.experimental.pallas import tpu as pltpu

N = 10000
D = 128
H = 64
DA = D + 2
BI = 400
NBLK = N // BI

PREC_DIST = lax.Precision.HIGHEST
PREC_AGG = lax.Precision.HIGHEST
PREC_SMALL = lax.Precision.HIGHEST


def _prep_kernel(x_ref, a_ref, b_ref):
    x = x_ref[...]
    sq = jnp.sum(x * x, axis=1, keepdims=True)
    ones = jnp.ones_like(sq)
    a_ref[...] = jnp.concatenate([-2.0 * x, sq, ones], axis=1)
    b_ref[...] = jnp.concatenate([x, ones, sq], axis=1)


def _layer_kernel(a_ref, b_ref, hf_ref, hb_ref, w_ref, bias_ref, eps_ref, out_ref):
    d2 = lax.dot_general(
        a_ref[...], b_ref[...], (((1,), (1,)), ((), ())), precision=PREC_DIST
    )
    maskf = (d2 < 1.0).astype(jnp.float32)
    agg = lax.dot_general(
        maskf, hf_ref[...], (((1,), (0,)), ((), ())), precision=PREC_AGG
    )
    z = eps_ref[0, 0] * hb_ref[...] + agg
    y = lax.dot_general(
        z, w_ref[...], (((1,), (1,)), ((), ())), precision=PREC_SMALL
    )
    out_ref[...] = jnp.maximum(y + bias_ref[...], 0.0)


def _head_kernel(h_ref, w1_ref, b1_ref, w2_ref, b2_ref, out_ref):
    g = jnp.mean(h_ref[...], axis=0, keepdims=True)
    g1 = lax.dot_general(
        g, w1_ref[...], (((1,), (1,)), ((), ())), precision=PREC_SMALL
    )
    g1 = jnp.maximum(g1 + b1_ref[...], 0.0)
    g2 = lax.dot_general(
        g1, w2_ref[...], (((1,), (1,)), ((), ())), precision=PREC_SMALL
    )
    out_ref[...] = g2 + b2_ref[...]


def _full(shape):
    return pl.BlockSpec(shape, lambda i: (0, 0))


def _layer(a, b, h, w, bias, eps, interpret=False):
    hin = h.shape[1]
    return pl.pallas_call(
        _layer_kernel,
        grid=(NBLK,),
        in_specs=[
            pl.BlockSpec((BI, DA), lambda i: (i, 0)),
            _full((N, DA)),
            _full((N, hin)),
            pl.BlockSpec((BI, hin), lambda i: (i, 0)),
            _full((H, hin)),
            _full((1, H)),
            pl.BlockSpec(memory_space=pltpu.SMEM),
        ],
        out_specs=pl.BlockSpec((BI, H), lambda i: (i, 0)),
        out_shape=jax.ShapeDtypeStruct((N, H), jnp.float32),
        interpret=interpret,
    )(a, b, h, h, w, bias, eps)


@functools.partial(jax.jit, static_argnames=("interpret",))
def _run(x, eps1, W1, b1, eps2, W2, b2, eps3, W3, b3, fc1_w, fc1_b, fc2_w, fc2_b,
         interpret=False):
    a, b = pl.pallas_call(
        _prep_kernel,
        out_shape=[
            jax.ShapeDtypeStruct((N, DA), jnp.float32),
            jax.ShapeDtypeStruct((N, DA), jnp.float32),
        ],
        interpret=interpret,
    )(x)

    e1 = jnp.reshape(eps1, (1, 1))
    e2 = jnp.reshape(eps2, (1, 1))
    e3 = jnp.reshape(eps3, (1, 1))
    h = _layer(a, b, x, W1, b1[None, :], e1, interpret)
    h = _layer(a, b, h, W2, b2[None, :], e2, interpret)
    h = _layer(a, b, h, W3, b3[None, :], e3, interpret)

    _full0 = lambda shape: pl.BlockSpec(shape, lambda: (0, 0))
    return pl.pallas_call(
        _head_kernel,
        in_specs=[
            _full0((N, H)),
            _full0((16, H)),
            _full0((1, 16)),
            _full0((D, 16)),
            _full0((1, D)),
        ],
        out_specs=_full0((1, D)),
        out_shape=jax.ShapeDtypeStruct((1, D), jnp.float32),
        interpret=interpret,
    )(h, fc1_w, fc1_b[None, :], fc2_w, fc2_b[None, :])


def kernel(x, eps1, W1, b1, eps2, W2, b2, eps3, W3, b3, fc1_w, fc1_b, fc2_w, fc2_b):
    return _run(x, eps1, W1, b1, eps2, W2, b2, eps3, W3, b3,
                fc1_w, fc1_b, fc2_w, fc2_b)


# speedup vs baseline: 5.0483x; 5.0483x over previous
import functools

import jax
import jax.numpy as jnp
from jax import lax
from jax.experimental import pallas as pl
from jax.experimental.pallas import tpu as pltpu

N = 10000
D = 128
H = 64
DA = D + 2
BI = 400
NBLK = N // BI

PREC_DIST = lax.Precision.HIGHEST
PREC_AGG = lax.Precision.HIGHEST
PREC_SMALL = lax.Precision.HIGHEST


def _prep_kernel(x_ref, a_ref, b_ref):
    x = x_ref[...]
    sq = jnp.sum(x * x, axis=1, keepdims=True)
    ones = jnp.ones_like(sq)
    a_ref[...] = jnp.concatenate([-2.0 * x, sq, ones], axis=1).astype(jnp.bfloat16)
    b_ref[...] = jnp.concatenate([x, ones, sq], axis=1).astype(jnp.bfloat16)


def _layer_kernel(a_ref, b_ref, hf_ref, hb_ref, w_ref, bias_ref, eps_ref, out_ref):
    d2 = lax.dot_general(
        a_ref[...], b_ref[...], (((1,), (1,)), ((), ())),
        preferred_element_type=jnp.float32,
    )
    maskf = (d2 < 1.0).astype(jnp.float32).astype(jnp.bfloat16)
    agg = lax.dot_general(
        maskf, hf_ref[...].astype(jnp.bfloat16), (((1,), (0,)), ((), ())),
        preferred_element_type=jnp.float32,
    )
    z = eps_ref[0, 0] * hb_ref[...] + agg
    y = lax.dot_general(
        z, w_ref[...], (((1,), (1,)), ((), ())), precision=PREC_SMALL
    )
    out_ref[...] = jnp.maximum(y + bias_ref[...], 0.0)


def _head_kernel(h_ref, w1_ref, b1_ref, w2_ref, b2_ref, out_ref):
    g = jnp.mean(h_ref[...], axis=0, keepdims=True)
    g1 = lax.dot_general(
        g, w1_ref[...], (((1,), (1,)), ((), ())), precision=PREC_SMALL
    )
    g1 = jnp.maximum(g1 + b1_ref[...], 0.0)
    g2 = lax.dot_general(
        g1, w2_ref[...], (((1,), (1,)), ((), ())), precision=PREC_SMALL
    )
    out_ref[...] = g2 + b2_ref[...]


def _full(shape):
    return pl.BlockSpec(shape, lambda i: (0, 0))


def _layer(a, b, h, w, bias, eps, interpret=False):
    hin = h.shape[1]
    return pl.pallas_call(
        _layer_kernel,
        grid=(NBLK,),
        in_specs=[
            pl.BlockSpec((BI, DA), lambda i: (i, 0)),
            _full((N, DA)),
            _full((N, hin)),
            pl.BlockSpec((BI, hin), lambda i: (i, 0)),
            _full((H, hin)),
            _full((1, H)),
            pl.BlockSpec(memory_space=pltpu.SMEM),
        ],
        out_specs=pl.BlockSpec((BI, H), lambda i: (i, 0)),
        out_shape=jax.ShapeDtypeStruct((N, H), jnp.float32),
        interpret=interpret,
    )(a, b, h, h, w, bias, eps)


@functools.partial(jax.jit, static_argnames=("interpret",))
def _run(x, eps1, W1, b1, eps2, W2, b2, eps3, W3, b3, fc1_w, fc1_b, fc2_w, fc2_b,
         interpret=False):
    a, b = pl.pallas_call(
        _prep_kernel,
        out_shape=[
            jax.ShapeDtypeStruct((N, DA), jnp.bfloat16),
            jax.ShapeDtypeStruct((N, DA), jnp.bfloat16),
        ],
        interpret=interpret,
    )(x)

    e1 = jnp.reshape(eps1, (1, 1))
    e2 = jnp.reshape(eps2, (1, 1))
    e3 = jnp.reshape(eps3, (1, 1))
    h = _layer(a, b, x, W1, b1[None, :], e1, interpret)
    h = _layer(a, b, h, W2, b2[None, :], e2, interpret)
    h = _layer(a, b, h, W3, b3[None, :], e3, interpret)

    _full0 = lambda shape: pl.BlockSpec(shape, lambda: (0, 0))
    return pl.pallas_call(
        _head_kernel,
        in_specs=[
            _full0((N, H)),
            _full0((16, H)),
            _full0((1, 16)),
            _full0((D, 16)),
            _full0((1, D)),
        ],
        out_specs=_full0((1, D)),
        out_shape=jax.ShapeDtypeStruct((1, D), jnp.float32),
        interpret=interpret,
    )(h, fc1_w, fc1_b[None, :], fc2_w, fc2_b[None, :])


def kernel(x, eps1, W1, b1, eps2, W2, b2, eps3, W3, b3, fc1_w, fc1_b, fc2_w, fc2_b):
    return _run(x, eps1, W1, b1, eps2, W2, b2, eps3, W3, b3,
                fc1_w, fc1_b, fc2_w, fc2_b)


# speedup vs baseline: 6.4037x; 1.2685x over previous
import functools

import jax
import jax.numpy as jnp
from jax import lax
from jax.experimental import pallas as pl
from jax.experimental.pallas import tpu as pltpu

N = 10000
NP = 10240
D = 128
H = 64
DA = D + 2
T = 2048
NT = NP // T
NPAIR = NT * (NT + 1) // 2
BIG = 1e9

_ROW_START = [0]
for _i in range(NT):
    _ROW_START.append(_ROW_START[-1] + NT - _i)


def _pair_ij(p):
    i = jnp.zeros_like(p)
    j = p
    for _r in range(1, NT):
        hit = p >= _ROW_START[_r]
        i = jnp.where(hit, _r, i)
        j = jnp.where(hit, _r + p - _ROW_START[_r], j)
    return i, j


def _prep_kernel(x_ref, a_ref, b_ref, xp_ref):
    x = x_ref[...]
    sq = jnp.sum(x * x, axis=1, keepdims=True)
    ones = jnp.ones_like(sq)
    zeros = jnp.zeros((NP - N, 1), jnp.float32)
    a = jnp.concatenate([-2.0 * x, sq, ones], axis=1)
    a_pad = jnp.concatenate(
        [jnp.zeros((NP - N, D), jnp.float32), jnp.full((NP - N, 1), BIG), zeros],
        axis=1)
    b = jnp.concatenate([x, ones, sq], axis=1)
    b_pad = jnp.concatenate(
        [jnp.zeros((NP - N, D), jnp.float32), zeros + 1.0,
         jnp.full((NP - N, 1), BIG)], axis=1)
    a_ref[...] = jnp.concatenate([a, a_pad], axis=0).astype(jnp.bfloat16)
    b_ref[...] = jnp.concatenate([b, b_pad], axis=0).astype(jnp.bfloat16)
    xp_ref[...] = jnp.concatenate([x, jnp.zeros((NP - N, D), jnp.float32)],
                                  axis=0)


def _agg_kernel(a_ref, b_ref, hf_ref, out_ref, mask_ref):
    p = pl.program_id(0)

    d2 = lax.dot_general(
        a_ref[...], b_ref[...], (((1,), (1,)), ((), ())),
        preferred_element_type=jnp.float32,
    )
    maskf = (d2 < 1.0).astype(jnp.float32).astype(jnp.float8_e4m3fn)
    mask_ref[pl.ds(lax.rem(p, 2), 1), :, :] = maskf[None]

    @pl.when(p == 0)
    def _():
        out_ref[...] = jnp.zeros_like(out_ref)

    q = jnp.maximum(p - 1, 0)
    iq, jq = _pair_ij(q)
    live = p > 0
    offd = live & (iq != jq)
    prev = mask_ref[pl.ds(lax.rem(q, 2), 1), :, :][0]

    hjf = hf_ref[pl.ds(jq * T, T), :]
    mj = jnp.max(jnp.abs(hjf))
    sj = jnp.exp2(jnp.floor(jnp.log2(128.0 / jnp.maximum(mj, 1e-30))))
    hj = (hjf * sj).astype(jnp.float8_e4m3fn)
    part1 = lax.dot_general(
        prev, hj, (((1,), (0,)), ((), ())),
        preferred_element_type=jnp.float32,
    )
    o1 = out_ref[pl.ds(iq * T, T), :]
    out_ref[pl.ds(iq * T, T), :] = jnp.where(live, o1 + (1.0 / sj) * part1, o1)

    hif = hf_ref[pl.ds(iq * T, T), :]
    mi = jnp.max(jnp.abs(hif))
    si = jnp.exp2(jnp.floor(jnp.log2(128.0 / jnp.maximum(mi, 1e-30))))
    hi = (hif * si).astype(jnp.float8_e4m3fn)
    part2 = lax.dot_general(
        prev, hi, (((0,), (0,)), ((), ())),
        preferred_element_type=jnp.float32,
    )
    o2 = out_ref[pl.ds(jq * T, T), :]
    out_ref[pl.ds(jq * T, T), :] = jnp.where(offd, o2 + (1.0 / si) * part2, o2)


def _apply_kernel(hf_ref, agg_ref, w_ref, bias_ref, eps_ref, out_ref):
    z = eps_ref[0, 0] * hf_ref[...] + agg_ref[...]
    y = lax.dot_general(
        z, w_ref[...], (((1,), (1,)), ((), ())),
        precision=lax.Precision.DEFAULT,
    )
    out_ref[...] = jnp.maximum(y + bias_ref[...], 0.0)


def _head_kernel(h_ref, w1_ref, b1_ref, w2_ref, b2_ref, out_ref):
    g = jnp.mean(h_ref[0:N, :], axis=0, keepdims=True)
    g1 = lax.dot_general(
        g, w1_ref[...], (((1,), (1,)), ((), ())),
        precision=lax.Precision.HIGHEST,
    )
    g1 = jnp.maximum(g1 + b1_ref[...], 0.0)
    g2 = lax.dot_general(
        g1, w2_ref[...], (((1,), (1,)), ((), ())),
        precision=lax.Precision.HIGHEST,
    )
    out_ref[...] = g2 + b2_ref[...]


def _layer(a, b, h, w, bias, eps, interpret=False):
    hin = h.shape[1]
    def _a_map(p):
        i, _ = _pair_ij(jnp.minimum(p, NPAIR - 1))
        return (i, 0)

    def _b_map(p):
        _, j = _pair_ij(jnp.minimum(p, NPAIR - 1))
        return (j, 0)

    agg = pl.pallas_call(
        _agg_kernel,
        grid=(NPAIR + 1,),
        in_specs=[
            pl.BlockSpec((T, DA), _a_map),
            pl.BlockSpec((T, DA), _b_map),
            pl.BlockSpec((NP, hin), lambda p: (0, 0)),
        ],
        out_specs=pl.BlockSpec((NP, hin), lambda p: (0, 0)),
        out_shape=jax.ShapeDtypeStruct((NP, hin), jnp.float32),
        scratch_shapes=[
            pltpu.VMEM((2, T, T), jnp.float8_e4m3fn),
        ],
        interpret=interpret,
    )(a, b, h)
    _full0 = lambda shape: pl.BlockSpec(shape, lambda: (0, 0))
    return pl.pallas_call(
        _apply_kernel,
        in_specs=[
            _full0((NP, hin)),
            _full0((NP, hin)),
            _full0((H, hin)),
            _full0((1, H)),
            pl.BlockSpec(memory_space=pltpu.SMEM),
        ],
        out_specs=_full0((NP, H)),
        out_shape=jax.ShapeDtypeStruct((NP, H), jnp.float32),
        interpret=interpret,
    )(h, agg, w, bias, eps)


@functools.partial(jax.jit, static_argnames=("interpret",))
def _run(x, eps1, W1, b1, eps2, W2, b2, eps3, W3, b3, fc1_w, fc1_b, fc2_w, fc2_b,
         interpret=False):
    a, b, xp = pl.pallas_call(
        _prep_kernel,
        out_shape=[
            jax.ShapeDtypeStruct((NP, DA), jnp.bfloat16),
            jax.ShapeDtypeStruct((NP, DA), jnp.bfloat16),
            jax.ShapeDtypeStruct((NP, D), jnp.float32),
        ],
        interpret=interpret,
    )(x)

    e1 = jnp.reshape(eps1, (1, 1))
    e2 = jnp.reshape(eps2, (1, 1))
    e3 = jnp.reshape(eps3, (1, 1))
    h = _layer(a, b, xp, W1, b1[None, :], e1, interpret)
    h = _layer(a, b, h, W2, b2[None, :], e2, interpret)
    h = _layer(a, b, h, W3, b3[None, :], e3, interpret)

    _full0 = lambda shape: pl.BlockSpec(shape, lambda: (0, 0))
    return pl.pallas_call(
        _head_kernel,
        in_specs=[
            _full0((NP, H)),
            _full0((16, H)),
            _full0((1, 16)),
            _full0((D, 16)),
            _full0((1, D)),
        ],
        out_specs=_full0((1, D)),
        out_shape=jax.ShapeDtypeStruct((1, D), jnp.float32),
        interpret=interpret,
    )(h, fc1_w, fc1_b[None, :], fc2_w, fc2_b[None, :])


def kernel(x, eps1, W1, b1, eps2, W2, b2, eps3, W3, b3, fc1_w, fc1_b, fc2_w, fc2_b):
    return _run(x, eps1, W1, b1, eps2, W2, b2, eps3, W3, b3,
                fc1_w, fc1_b, fc2_w, fc2_b)
